# baseline (speedup 1.0000x reference)
import functools

import jax
import jax.numpy as jnp
from jax.experimental import pallas as pl
from jax.experimental.pallas import tpu as pltpu

_BN = 1024
_NS = 8
_W = _NS * _BN


def _pfc_kernel(a_ref, *refs):
    w_refs = refs[:_NS]
    o_ref = refs[_NS]
    a = a_ref[...].astype(jnp.bfloat16)
    for j, w_ref in enumerate(w_refs):
        w = w_ref[...].astype(jnp.bfloat16)
        o_ref[:, j * _BN:(j + 1) * _BN] = jax.lax.dot_general(
            a, w,
            dimension_numbers=(((1,), (1,)), ((), ())),
            preferred_element_type=jnp.float32,
        )


def _w_index_map(j, last_block, i):
    return jnp.minimum(_NS * i + j, last_block), 0


def kernel(total_features, norm_weight):
    b, k = total_features.shape
    n = norm_weight.shape[0]
    last_block = pl.cdiv(n, _BN) - 1
    grid = (pl.cdiv(n, _W),)
    w_specs = [
        pl.BlockSpec((_BN, k), functools.partial(_w_index_map, j, last_block))
        for j in range(_NS)
    ]
    return pl.pallas_call(
        _pfc_kernel,
        grid=grid,
        in_specs=[pl.BlockSpec((b, k), lambda i: (0, 0))] + w_specs,
        out_specs=pl.BlockSpec((b, _W), lambda i: (0, i)),
        out_shape=jax.ShapeDtypeStruct((b, n), jnp.float32),
        compiler_params=pltpu.CompilerParams(
            dimension_semantics=("arbitrary",),
        ),
    )(total_features, *([norm_weight] * _NS))


# speedup vs baseline: 1.0283x; 1.0283x over previous
import functools

import jax
import jax.numpy as jnp
from jax.experimental import pallas as pl
from jax.experimental.pallas import tpu as pltpu

_BN = 512
_NS = 8
_W = _NS * _BN


def _pfc_kernel(a_ref, *refs):
    w_refs = refs[:_NS]
    o_ref = refs[_NS]
    a = a_ref[...].astype(jnp.bfloat16)
    for j, w_ref in enumerate(w_refs):
        w = w_ref[...].astype(jnp.bfloat16)
        o_ref[:, j * _BN:(j + 1) * _BN] = jax.lax.dot_general(
            a, w,
            dimension_numbers=(((1,), (1,)), ((), ())),
            preferred_element_type=jnp.float32,
        )


def _w_index_map(j, last_block, i):
    return jnp.minimum(_NS * i + j, last_block), 0


def kernel(total_features, norm_weight):
    b, k = total_features.shape
    n = norm_weight.shape[0]
    last_block = pl.cdiv(n, _BN) - 1
    grid = (pl.cdiv(n, _W),)
    w_specs = [
        pl.BlockSpec((_BN, k), functools.partial(_w_index_map, j, last_block))
        for j in range(_NS)
    ]
    return pl.pallas_call(
        _pfc_kernel,
        grid=grid,
        in_specs=[pl.BlockSpec((b, k), lambda i: (0, 0))] + w_specs,
        out_specs=pl.BlockSpec((b, _W), lambda i: (0, i)),
        out_shape=jax.ShapeDtypeStruct((b, n), jnp.float32),
        compiler_params=pltpu.CompilerParams(
            dimension_semantics=("arbitrary",),
        ),
    )(total_features, *([norm_weight] * _NS))
